# Initial kernel scaffold; baseline (speedup 1.0000x reference)
import jax
import jax.numpy as jnp
from jax.experimental import pallas as pl


def kernel(vdata, edata, sender_ids, receiver_ids, W, b):
    raise NotImplementedError("write your pallas kernel here")


# speedup vs baseline: 1709.1748x; 1709.1748x over previous
import functools

import jax
import jax.numpy as jnp
from jax import lax
from jax.experimental import pallas as pl
from jax.experimental.pallas import tpu as pltpu
from jax.experimental.pallas import tpu_sc as plsc

NW = 32
C = 1000
G = 125
NG = C // G
BE = 8000


def _proj_tables_body(vd_ref, ws_ref, wr_ref, ps_ref, pr_ref):
    vd = vd_ref[...]
    ps_ref[...] = jnp.dot(vd, ws_ref[...], preferred_element_type=jnp.float32)
    pr_ref[...] = jnp.dot(vd, wr_ref[...], preferred_element_type=jnp.float32)


def _eproj_body(ed_ref, we_ref, b_ref, out_ref):
    out_ref[...] = (
        jnp.dot(ed_ref[...], we_ref[...], preferred_element_type=jnp.float32)
        + b_ref[...]
    )


def _sc_body(ps_hbm, pr_hbm, eproj_hbm, sids_hbm, rids_hbm, out_hbm,
             sidx_v, ridx_v, gs_v, gr_v, acc_v, sem_s, sem_r, sem_e):
    E = out_hbm.shape[0]
    ew = E // NW
    nch = ew // C
    wid = lax.axis_index("s") * 2 + lax.axis_index("c")

    for ch in range(nch):
        base = pl.multiple_of(wid * ew + ch * C, 8)
        row0 = pl.multiple_of((wid * ew + ch * C) // G, 8)
        pltpu.sync_copy(sids_hbm.at[pl.ds(row0, NG)], sidx_v)
        pltpu.sync_copy(rids_hbm.at[pl.ds(row0, NG)], ridx_v)
        cps = [pltpu.async_copy(eproj_hbm.at[pl.ds(base, C)], acc_v, sem_e)]
        for j in range(NG):
            cps.append(pltpu.async_copy(
                ps_hbm.at[sidx_v.at[j]], gs_v.at[j], sem_s))
            cps.append(pltpu.async_copy(
                pr_hbm.at[ridx_v.at[j]], gr_v.at[j], sem_r))
        for cp in cps:
            cp.wait()

        for j in range(NG):
            @plsc.parallel_loop(0, G, 1, unroll=5)
            def _row(i):
                acc_v[j * G + i, :] = jnp.maximum(
                    acc_v[j * G + i, :] + gs_v[j, i, :] + gr_v[j, i, :], 0.0)

        pltpu.sync_copy(acc_v, out_hbm.at[pl.ds(base, C)])


def kernel(vdata, edata, sender_ids, receiver_ids, W, b):
    Bn, N, DV = vdata.shape
    _, E, DE = edata.shape
    DOUT = W.shape[1]
    assert E % (NW * C) == 0 and C % G == 0 and E % BE == 0

    vd = vdata.reshape(N, DV)
    ed = edata.reshape(E, DE)
    sid = sender_ids.reshape(E // G, G)
    rid = receiver_ids.reshape(E // G, G)
    we = W[:DE]
    ws = W[DE:DE + DV]
    wr = W[DE + DV:]

    ps, pr = pl.pallas_call(
        _proj_tables_body,
        out_shape=[jax.ShapeDtypeStruct((N, DOUT), jnp.float32)] * 2,
    )(vd, ws, wr)

    eproj = pl.pallas_call(
        _eproj_body,
        grid=(E // BE,),
        in_specs=[
            pl.BlockSpec((BE, DE), lambda i: (i, 0)),
            pl.BlockSpec((DE, DOUT), lambda i: (0, 0)),
            pl.BlockSpec((1, DOUT), lambda i: (0, 0)),
        ],
        out_specs=pl.BlockSpec((BE, DOUT), lambda i: (i, 0)),
        out_shape=jax.ShapeDtypeStruct((E, DOUT), jnp.float32),
    )(ed, we, b.reshape(1, DOUT))

    mesh = plsc.VectorSubcoreMesh(core_axis_name="c", subcore_axis_name="s")
    sc_call = pl.kernel(
        _sc_body,
        out_type=jax.ShapeDtypeStruct((E, DOUT), jnp.float32),
        mesh=mesh,
        compiler_params=pltpu.CompilerParams(use_tc_tiling_on_sc=False),
        scratch_types=[
            pltpu.VMEM((NG, G), jnp.int32),
            pltpu.VMEM((NG, G), jnp.int32),
            pltpu.VMEM((NG, G, DOUT), jnp.float32),
            pltpu.VMEM((NG, G, DOUT), jnp.float32),
            pltpu.VMEM((C, DOUT), jnp.float32),
            pltpu.SemaphoreType.DMA,
            pltpu.SemaphoreType.DMA,
            pltpu.SemaphoreType.DMA,
        ],
    )
    out = sc_call(ps, pr, eproj, sid, rid)
    return out.reshape(Bn, E, DOUT)


# speedup vs baseline: 1833.8377x; 1.0729x over previous
import functools

import jax
import jax.numpy as jnp
from jax import lax
from jax.experimental import pallas as pl
from jax.experimental.pallas import tpu as pltpu
from jax.experimental.pallas import tpu_sc as plsc

NW = 32
EP = 327680
CE = 1024
CP = CE // 8
G = 128
NG = CE // G
BEP = 1024


def _tc_body(vd_ref, ws_ref, wr_ref, edp_ref, wblk_ref, bp_ref,
             ps_ref, pr_ref, ep_ref):
    @pl.when(pl.program_id(0) == 0)
    def _tables():
        vd = vd_ref[...]
        ps_ref[...] = jnp.dot(vd, ws_ref[...],
                              preferred_element_type=jnp.float32)
        pr_ref[...] = jnp.dot(vd, wr_ref[...],
                              preferred_element_type=jnp.float32)

    ep_ref[...] = (
        jnp.dot(edp_ref[...], wblk_ref[...],
                preferred_element_type=jnp.float32)
        + bp_ref[...]
    )


def _sc_body(ps_hbm, pr_hbm, ep_hbm, sids_hbm, rids_hbm, out_hbm,
             sidx_v, ridx_v, gs_v, gr_v, acc_v, sem_s, sem_r, sem_e):
    ew = EP // NW
    nch = ew // CE
    wid = lax.axis_index("s") * 2 + lax.axis_index("c")

    for ch in range(nch):
        ebase = wid * ew + ch * CE
        row0 = pl.multiple_of(ebase // G, 8)
        prow0 = pl.multiple_of(ebase // 8, 8)
        pltpu.sync_copy(sids_hbm.at[pl.ds(row0, NG)], sidx_v)
        pltpu.sync_copy(rids_hbm.at[pl.ds(row0, NG)], ridx_v)
        cps = [pltpu.async_copy(ep_hbm.at[pl.ds(prow0, CP)], acc_v, sem_e)]
        for j in range(NG):
            cps.append(pltpu.async_copy(
                ps_hbm.at[sidx_v.at[j]], gs_v.at[pl.ds(j * G, G)], sem_s))
            cps.append(pltpu.async_copy(
                pr_hbm.at[ridx_v.at[j]], gr_v.at[pl.ds(j * G, G)], sem_r))
        for cp in cps:
            cp.wait()

        @plsc.parallel_loop(0, CP, 1, unroll=2)
        def _row(g):
            for k in range(8):
                e = g * 8 + k
                acc_v[g, pl.ds(16 * k, 16)] = jnp.maximum(
                    acc_v[g, pl.ds(16 * k, 16)] + gs_v[e, :] + gr_v[e, :],
                    0.0)

        pltpu.sync_copy(acc_v, out_hbm.at[pl.ds(prow0, CP)])


def kernel(vdata, edata, sender_ids, receiver_ids, W, b):
    Bn, N, DV = vdata.shape
    _, E, DE = edata.shape
    DOUT = W.shape[1]

    vd = vdata.reshape(N, DV)
    edp = edata.reshape(E * DE // 128, 128)
    pad = EP - E
    sid = jnp.pad(sender_ids.reshape(E), (0, pad)).reshape(EP // G, G)
    rid = jnp.pad(receiver_ids.reshape(E), (0, pad)).reshape(EP // G, G)
    we = W[:DE]
    ws = W[DE:DE + DV]
    wr = W[DE + DV:]
    wblk = jax.scipy.linalg.block_diag(*([we] * 8))
    bp = jnp.tile(b, 8).reshape(1, 128)

    npk = EP // 8
    ps, pr, ep = pl.pallas_call(
        _tc_body,
        grid=(npk // BEP,),
        in_specs=[
            pl.BlockSpec((N, DV), lambda i: (0, 0)),
            pl.BlockSpec((DV, DOUT), lambda i: (0, 0)),
            pl.BlockSpec((DV, DOUT), lambda i: (0, 0)),
            pl.BlockSpec((BEP, 128), lambda i: (i, 0)),
            pl.BlockSpec((128, 128), lambda i: (0, 0)),
            pl.BlockSpec((1, 128), lambda i: (0, 0)),
        ],
        out_specs=[
            pl.BlockSpec((N, DOUT), lambda i: (0, 0)),
            pl.BlockSpec((N, DOUT), lambda i: (0, 0)),
            pl.BlockSpec((BEP, 128), lambda i: (i, 0)),
        ],
        out_shape=[
            jax.ShapeDtypeStruct((N, DOUT), jnp.float32),
            jax.ShapeDtypeStruct((N, DOUT), jnp.float32),
            jax.ShapeDtypeStruct((npk, 128), jnp.float32),
        ],
    )(vd, ws, wr, edp, wblk, bp)

    mesh = plsc.VectorSubcoreMesh(core_axis_name="c", subcore_axis_name="s")
    sc_call = pl.kernel(
        _sc_body,
        out_type=jax.ShapeDtypeStruct((npk, 128), jnp.float32),
        mesh=mesh,
        compiler_params=pltpu.CompilerParams(use_tc_tiling_on_sc=False),
        scratch_types=[
            pltpu.VMEM((NG, G), jnp.int32),
            pltpu.VMEM((NG, G), jnp.int32),
            pltpu.VMEM((CE, 16), jnp.float32),
            pltpu.VMEM((CE, 16), jnp.float32),
            pltpu.VMEM((CP, 128), jnp.float32),
            pltpu.SemaphoreType.DMA,
            pltpu.SemaphoreType.DMA,
            pltpu.SemaphoreType.DMA,
        ],
    )
    outp = sc_call(ps, pr, ep, sid, rid)
    return outp.reshape(EP, DOUT)[:E].reshape(Bn, E, DOUT)


# speedup vs baseline: 2115.9104x; 1.1538x over previous
import functools

import jax
import jax.numpy as jnp
from jax import lax
from jax.experimental import pallas as pl
from jax.experimental.pallas import tpu as pltpu
from jax.experimental.pallas import tpu_sc as plsc

NW = 32
EP = 327680
CE = 1024
CP = CE // 8
G = 128
NG = CE // G
BEF = 1000


def _tables_body(vd_ref, ws_ref, wr_ref, ps_ref, pr_ref):
    vd = vd_ref[...]
    ps_ref[...] = jnp.dot(vd, ws_ref[...], preferred_element_type=jnp.float32)
    pr_ref[...] = jnp.dot(vd, wr_ref[...], preferred_element_type=jnp.float32)


def _sc_body(ps_hbm, pr_hbm, sids_hbm, rids_hbm, out_hbm,
             sidx_v, ridx_v, gs_v, gr_v, acc_v, sem_s, sem_r):
    ew = EP // NW
    nch = ew // CE
    wid = lax.axis_index("s") * 2 + lax.axis_index("c")

    for ch in range(nch):
        ebase = wid * ew + ch * CE
        row0 = pl.multiple_of(ebase // G, 8)
        prow0 = pl.multiple_of(ebase // 8, 8)
        pltpu.sync_copy(sids_hbm.at[pl.ds(row0, NG)], sidx_v)
        pltpu.sync_copy(rids_hbm.at[pl.ds(row0, NG)], ridx_v)
        cps = []
        for j in range(NG):
            cps.append(pltpu.async_copy(
                ps_hbm.at[sidx_v.at[j]], gs_v.at[pl.ds(j * G, G)], sem_s))
            cps.append(pltpu.async_copy(
                pr_hbm.at[ridx_v.at[j]], gr_v.at[pl.ds(j * G, G)], sem_r))
        for cp in cps:
            cp.wait()

        @plsc.parallel_loop(0, CP, 1, unroll=2)
        def _row(g):
            for k in range(8):
                e = g * 8 + k
                acc_v[g, pl.ds(16 * k, 16)] = gs_v[e, :] + gr_v[e, :]

        pltpu.sync_copy(acc_v, out_hbm.at[pl.ds(prow0, CP)])


def _final_body(edp_ref, wblk_ref, bp_ref, gsum_ref, out_ref):
    eproj = (
        jnp.dot(edp_ref[...], wblk_ref[...],
                preferred_element_type=jnp.float32)
        + bp_ref[...]
    )
    out_ref[...] = jnp.maximum(eproj + gsum_ref[...], 0.0)


def kernel(vdata, edata, sender_ids, receiver_ids, W, b):
    Bn, N, DV = vdata.shape
    _, E, DE = edata.shape
    DOUT = W.shape[1]

    vd = vdata.reshape(N, DV)
    edp = edata.reshape(E * DE // 128, 128)
    pad = EP - E
    sid = jnp.pad(sender_ids.reshape(E), (0, pad)).reshape(EP // G, G)
    rid = jnp.pad(receiver_ids.reshape(E), (0, pad)).reshape(EP // G, G)
    we = W[:DE]
    ws = W[DE:DE + DV]
    wr = W[DE + DV:]
    wblk = jax.scipy.linalg.block_diag(*([we] * 8))
    bp = jnp.tile(b, 8).reshape(1, 128)

    ps, pr = pl.pallas_call(
        _tables_body,
        out_shape=[jax.ShapeDtypeStruct((N, DOUT), jnp.float32)] * 2,
    )(vd, ws, wr)

    mesh = plsc.VectorSubcoreMesh(core_axis_name="c", subcore_axis_name="s")
    sc_call = pl.kernel(
        _sc_body,
        out_type=jax.ShapeDtypeStruct((EP // 8, 128), jnp.float32),
        mesh=mesh,
        compiler_params=pltpu.CompilerParams(use_tc_tiling_on_sc=False),
        scratch_types=[
            pltpu.VMEM((NG, G), jnp.int32),
            pltpu.VMEM((NG, G), jnp.int32),
            pltpu.VMEM((CE, 16), jnp.float32),
            pltpu.VMEM((CE, 16), jnp.float32),
            pltpu.VMEM((CP, 128), jnp.float32),
            pltpu.SemaphoreType.DMA,
            pltpu.SemaphoreType.DMA,
        ],
    )
    gsum = sc_call(ps, pr, sid, rid)

    npk = E * DE // 128
    outp = pl.pallas_call(
        _final_body,
        grid=(npk // BEF,),
        in_specs=[
            pl.BlockSpec((BEF, 128), lambda i: (i, 0)),
            pl.BlockSpec((128, 128), lambda i: (0, 0)),
            pl.BlockSpec((1, 128), lambda i: (0, 0)),
            pl.BlockSpec((BEF, 128), lambda i: (i, 0)),
        ],
        out_specs=pl.BlockSpec((BEF, 128), lambda i: (i, 0)),
        out_shape=jax.ShapeDtypeStruct((npk, 128), jnp.float32),
    )(edp, wblk, bp, gsum)
    return outp.reshape(Bn, E, DOUT)
